# baseline (speedup 1.0000x reference)
import jax
import jax.numpy as jnp
from jax import lax
from jax.experimental import pallas as pl
from jax.experimental.pallas import tpu as pltpu
from jax.experimental.pallas import tpu_sc as plsc

D = 64
SEQ = 50
NW = 32
SEQ_PER_CHUNK = 16
TOK_PER_CHUNK = SEQ_PER_CHUNK * SEQ


def _rsqrt(x):
    i = plsc.bitcast(x, jnp.int32)
    i = jnp.int32(0x5F3759DF) - lax.shift_right_logical(i, 1)
    y = plsc.bitcast(i, jnp.float32)
    for _ in range(2):
        y = y * (1.5 - 0.5 * x * y * y)
    return y


def _allsum(v):
    dnums = lax.GatherDimensionNumbers(
        offset_dims=(), collapsed_slice_dims=(0,), start_index_map=(0,))
    for step in (8, 4, 2, 1):
        perm = jnp.arange(16, dtype=jnp.int32) ^ step
        v = v + lax.gather(v, perm[:, None], dnums, slice_sizes=(1,),
                           mode=lax.GatherScatterMode.PROMISE_IN_BOUNDS)
    return v


def _body(x_hbm, tok_hbm, pos_hbm, lnw_hbm, lnb_hbm, out_hbm,
          idx_v, rows_v, pos_v, lnw_v, lnb_v, sem):
    cid = lax.axis_index("c")
    sid = lax.axis_index("s")
    wid = sid * 2 + cid
    n_chunks = x_hbm.shape[0] // (NW * TOK_PER_CHUNK)

    pltpu.sync_copy(pos_hbm.at[pl.ds(0, 56)], pos_v)
    pltpu.sync_copy(lnw_hbm, lnw_v)
    pltpu.sync_copy(lnb_hbm, lnb_v)

    lw = [lnw_v[pl.ds(k * 16, 16)] for k in range(4)]
    lb = [lnb_v[pl.ds(k * 16, 16)] for k in range(4)]

    def chunk_body(c, _):
        tok_base = pl.multiple_of((wid * n_chunks + c) * TOK_PER_CHUNK, 8)
        pltpu.sync_copy(x_hbm.at[pl.ds(tok_base, TOK_PER_CHUNK)], idx_v)
        pltpu.async_copy(tok_hbm.at[idx_v], rows_v, sem).wait()

        def s_body(s, _):
            p = [pos_v[s, pl.ds(k * 16, 16)] for k in range(4)]

            def q_body(q, _):
                t = q * SEQ + s
                e = [rows_v[t, pl.ds(k * 16, 16)] + p[k] for k in range(4)]
                tot = _allsum((e[0] + e[1]) + (e[2] + e[3]))
                tot2 = _allsum((e[0] * e[0] + e[1] * e[1])
                               + (e[2] * e[2] + e[3] * e[3]))
                mean = tot * (1.0 / D)
                var = tot2 * (1.0 / D) - mean * mean
                rstd = _rsqrt(var + 1e-5)
                for k in range(4):
                    rows_v[t, pl.ds(k * 16, 16)] = (
                        (e[k] - mean) * rstd * lw[k] + lb[k])
                return 0

            lax.fori_loop(0, SEQ_PER_CHUNK, q_body, 0, unroll=4)
            return 0

        lax.fori_loop(0, SEQ, s_body, 0)
        pltpu.sync_copy(rows_v, out_hbm.at[pl.ds(tok_base, TOK_PER_CHUNK)])
        return 0

    lax.fori_loop(0, n_chunks, chunk_body, 0)


def kernel(x, tok_table, pos_table, ln_w, ln_b):
    batch, seq = x.shape
    n_tok = batch * seq
    run = pl.kernel(
        _body,
        out_type=jax.ShapeDtypeStruct((n_tok, D), jnp.float32),
        mesh=plsc.VectorSubcoreMesh(core_axis_name="c", subcore_axis_name="s"),
        compiler_params=pltpu.CompilerParams(
            needs_layout_passes=False, use_tc_tiling_on_sc=False),
        scratch_types=[
            pltpu.VMEM((TOK_PER_CHUNK,), jnp.int32),
            pltpu.VMEM((TOK_PER_CHUNK, D), jnp.float32),
            pltpu.VMEM((56, D), jnp.float32),
            pltpu.VMEM((D,), jnp.float32),
            pltpu.VMEM((D,), jnp.float32),
            pltpu.SemaphoreType.DMA,
        ],
    )
    out = run(x.reshape(n_tok), tok_table, pos_table, ln_w, ln_b)
    return out.reshape(batch, seq, D)
